# baseline (speedup 1.0000x reference)
import functools

import numpy as np
import jax
import jax.numpy as jnp
from jax import lax
from jax.experimental import pallas as pl
from jax.experimental.pallas import tpu as pltpu
from jax.experimental.pallas import tpu_sc as plsc

D_MODEL = 64
SEQ_LEN = 200
BATCH = 1024
V_ROWS = 1000000
SCALE = np.float32(np.sqrt(D_MODEL))
NBUF_G = 3

_SPLIT_A = 128
_SPLIT_B = SEQ_LEN - _SPLIT_A


def _positional_encoding(length, depth):
    half = depth / 2
    positions = np.arange(length)[:, np.newaxis]
    depths = np.arange(half)[np.newaxis, :] / half
    angle_rates = 1 / 10000 ** depths
    angle_rads = positions * angle_rates
    pe = np.concatenate([np.sin(angle_rads), np.cos(angle_rads)], axis=-1)
    return pe.astype(np.float32)


_PE_NP = _positional_encoding(SEQ_LEN, D_MODEL)


@functools.cache
def _build_emb_lookup():
    info = plsc.get_sparse_core_info()
    nc, ns = info.num_cores, info.num_subcores
    nw = nc * ns
    seq_per_w = BATCH // nw
    n_idx = seq_per_w * SEQ_LEN
    mesh = plsc.VectorSubcoreMesh(core_axis_name="c", subcore_axis_name="s")

    @functools.partial(
        pl.kernel,
        mesh=mesh,
        out_type=jax.ShapeDtypeStruct((BATCH, SEQ_LEN // 2, 128), jnp.float32),
        scratch_types=[
            pltpu.VMEM((n_idx,), jnp.int32),
            pltpu.VMEM((n_idx,), jnp.int32),
            pltpu.VMEM((SEQ_LEN, D_MODEL), jnp.float32),
            [pltpu.VMEM((SEQ_LEN, 128), jnp.float32)] * NBUF_G,
            [pltpu.SemaphoreType.DMA] * NBUF_G,
            [pltpu.SemaphoreType.DMA] * NBUF_G,
        ],
        compiler_params=pltpu.CompilerParams(needs_layout_passes=False),
    )
    def _emb_lookup(idx_hbm, z_hbm, pe_hbm, out_hbm,
                    idx_v, iz_v, pe_v, bufs, gsems, osems):
        wid = lax.axis_index("s") * nc + lax.axis_index("c")
        w_base = wid * seq_per_w

        pltpu.sync_copy(pe_hbm, pe_v)
        pltpu.sync_copy(idx_hbm.at[pl.ds(w_base * SEQ_LEN, n_idx)], idx_v)

        def half_body(i, carry):
            sl = pl.ds(i * 16, 16)
            iz_v[sl] = lax.shift_right_logical(idx_v[sl], 1)
            return carry
        lax.fori_loop(0, n_idx // 16, half_body, 0, unroll=4)

        def fire_gather(s, b):
            base = s * SEQ_LEN
            c1 = pltpu.async_copy(
                z_hbm.at[iz_v.at[pl.ds(base, _SPLIT_A)]],
                bufs[b].at[pl.ds(0, _SPLIT_A)], gsems[b])
            c2 = pltpu.async_copy(
                z_hbm.at[iz_v.at[pl.ds(base + _SPLIT_A, _SPLIT_B)]],
                bufs[b].at[pl.ds(_SPLIT_A, _SPLIT_B)], gsems[b])
            return c1, c2

        pending_g = {}
        pending_o = {}
        for s in range(NBUF_G - 1):
            pending_g[s] = fire_gather(s, s)

        for s in range(seq_per_w):
            b = s % NBUF_G
            c1, c2 = pending_g.pop(s)
            c1.wait()
            c2.wait()
            buf = bufs[b]

            def row_body(rp, c, buf=buf, s=s):
                for q in range(2):
                    r = 2 * rp + q
                    rsplat = jnp.full((16,), s * SEQ_LEN + r, jnp.int32)
                    lsb16 = plsc.load_gather(idx_v, [rsplat]) & 1
                    csel = lsb16 > 0
                    for j in range(D_MODEL // 16):
                        lo = buf[r, pl.ds(j * 16, 16)]
                        hi = buf[r, pl.ds(64 + j * 16, 16)]
                        v = jnp.where(csel, hi, lo)
                        buf[rp, pl.ds(64 * q + j * 16, 16)] = (
                            v * SCALE + pe_v[r, pl.ds(j * 16, 16)])
                return c

            lax.fori_loop(0, SEQ_LEN // 2, row_body, 0)

            pending_o[s] = pltpu.async_copy(
                buf.at[pl.ds(0, SEQ_LEN // 2)], out_hbm.at[w_base + s],
                osems[b])

            nxt = s + NBUF_G - 1
            if nxt < seq_per_w:
                nb = nxt % NBUF_G
                if nxt - NBUF_G in pending_o:
                    pending_o.pop(nxt - NBUF_G).wait()
                pending_g[nxt] = fire_gather(nxt, nb)

        for s in sorted(pending_o):
            pending_o[s].wait()

    return _emb_lookup


def kernel(x, table):
    idx = x.reshape(-1).astype(jnp.int32)
    pe = jnp.asarray(_PE_NP)
    zp = table.reshape(V_ROWS // 2, 2 * D_MODEL)
    out = _build_emb_lookup()(idx, zp, pe)
    return out.reshape(BATCH, SEQ_LEN, D_MODEL)
